# Initial kernel scaffold; baseline (speedup 1.0000x reference)
import jax
import jax.numpy as jnp
from jax.experimental import pallas as pl


def kernel(nodes, edges, W_message, W_node, mlp_W1, mlp_b1, mlp_W2, mlp_b2, ln_scale, ln_bias, senders, receivers):
    raise NotImplementedError("write your pallas kernel here")


# speedup vs baseline: 3.6474x; 3.6474x over previous
import functools

import jax
import jax.numpy as jnp
from jax import lax
from jax.experimental import pallas as pl
from jax.experimental.pallas import tpu as pltpu
from jax.experimental.pallas import tpu_sc as plsc

N_NODES = 10000
N_EDGES = 320000
D_FEAT = 128
D_HALF = 64
D_EDGE = 16
LN_EPS = 1e-6

NC = 2
NS = 16
NW = NC * NS

CHUNK = 80
EDGES_PER_W = N_EDGES // NW
CHUNKS_PER_W = EDGES_PER_W // CHUNK
N_PAD = 10240
ROWS_PER_TILE = N_PAD // NS


def _sc_body(nodesL_hbm, nodesR_hbm, edges_hbm, send_hbm, recv_hbm, z_hbm,
             ze_hbm, pnL_hbm, pnR_hbm, pe_hbm,
             sidx, ridx, rows, erows, acc, eacc, sem):
    cid = lax.axis_index("c")
    sid = lax.axis_index("s")
    w = cid * NS + sid
    mine = pl.ds(sid * ROWS_PER_TILE, ROWS_PER_TILE)

    pltpu.sync_copy(z_hbm, acc.at[mine])
    pltpu.sync_copy(ze_hbm, eacc.at[mine])

    pltpu.sync_copy(send_hbm.at[w], sidx)
    pltpu.sync_copy(recv_hbm.at[w], ridx)

    plsc.subcore_barrier()

    def step0(j, carry):
        pltpu.async_copy(nodesL_hbm.at[sidx.at[j]], rows, sem).wait()
        pltpu.sync_copy(
            edges_hbm.at[pl.ds(w * EDGES_PER_W + j * CHUNK, CHUNK)], erows)
        pltpu.sync_copy(rows, acc.at[ridx.at[j]], add=True)
        pltpu.sync_copy(erows, eacc.at[ridx.at[j]], add=True)
        return carry

    lax.fori_loop(0, CHUNKS_PER_W, step0, 0)
    plsc.subcore_barrier()

    pltpu.sync_copy(acc.at[mine], pnL_hbm.at[cid].at[mine])
    pltpu.sync_copy(eacc.at[mine], pe_hbm.at[cid].at[mine])
    pltpu.sync_copy(z_hbm, acc.at[mine])
    plsc.subcore_barrier()

    def step1(j, carry):
        pltpu.async_copy(nodesR_hbm.at[sidx.at[j]], rows, sem).wait()
        pltpu.sync_copy(rows, acc.at[ridx.at[j]], add=True)
        return carry

    lax.fori_loop(0, CHUNKS_PER_W, step1, 0)
    plsc.subcore_barrier()

    pltpu.sync_copy(acc.at[mine], pnR_hbm.at[cid].at[mine])


_sc_segsum = functools.partial(
    pl.kernel,
    out_type=[
        jax.ShapeDtypeStruct((NC, N_PAD, D_HALF), jnp.float32),
        jax.ShapeDtypeStruct((NC, N_PAD, D_HALF), jnp.float32),
        jax.ShapeDtypeStruct((NC, N_PAD, D_EDGE), jnp.float32),
    ],
    mesh=plsc.VectorSubcoreMesh(core_axis_name="c", subcore_axis_name="s"),
    scratch_types=[
        pltpu.VMEM((CHUNKS_PER_W, CHUNK), jnp.int32),
        pltpu.VMEM((CHUNKS_PER_W, CHUNK), jnp.int32),
        pltpu.VMEM((CHUNK, D_HALF), jnp.float32),
        pltpu.VMEM((CHUNK, D_EDGE), jnp.float32),
        pltpu.VMEM_SHARED((N_PAD, D_HALF), jnp.float32),
        pltpu.VMEM_SHARED((N_PAD, D_EDGE), jnp.float32),
        pltpu.SemaphoreType.DMA,
    ],
    compiler_params=pltpu.CompilerParams(use_tc_tiling_on_sc=False),
)(_sc_body)


def _tc_body(nodes, pl0, pl1, pr0, pr1, pe0, pe1,
             wm_a, wm_b, wm_bot, w_node, w1_top, w1_bot, w2,
             b1, b2, g, b, out):
    f32 = jnp.float32
    agg = (jnp.dot(pl0[0] + pl1[0], wm_a[...], preferred_element_type=f32)
           + jnp.dot(pr0[0] + pr1[0], wm_b[...], preferred_element_type=f32)
           + jnp.dot(pe0[0] + pe1[0], wm_bot[...], preferred_element_type=f32))
    x = nodes[...]
    h = jnp.maximum(
        jnp.dot(x, w1_top[...], preferred_element_type=f32)
        + jnp.dot(agg, w1_bot[...], preferred_element_type=f32)
        + b1[...], 0.0)
    pre = (jnp.dot(h, w2[...], preferred_element_type=f32) + b2[...]
           + jnp.dot(x, w_node[...], preferred_element_type=f32))
    mean = jnp.mean(pre, axis=-1, keepdims=True)
    c = pre - mean
    var = jnp.mean(c * c, axis=-1, keepdims=True)
    out[...] = c * lax.rsqrt(var + LN_EPS) * g[...] + b[...]


def kernel(nodes, edges, W_message, W_node, mlp_W1, mlp_b1, mlp_W2, mlp_b2,
           ln_scale, ln_bias, senders, receivers):
    nodesL = nodes[:, :D_HALF]
    nodesR = nodes[:, D_HALF:]
    send3d = senders.reshape(NW, CHUNKS_PER_W, CHUNK)
    recv3d = receivers.reshape(NW, CHUNKS_PER_W, CHUNK)
    z = jnp.zeros((ROWS_PER_TILE, D_HALF), jnp.float32)
    ze = jnp.zeros((ROWS_PER_TILE, D_EDGE), jnp.float32)

    pnL, pnR, pe = _sc_segsum(nodesL, nodesR, edges, send3d, recv3d, z, ze)

    R = 1000
    grid = (N_NODES // R,)
    full = lambda shape: pl.BlockSpec(shape, lambda i: (0, 0))
    out = pl.pallas_call(
        _tc_body,
        grid=grid,
        in_specs=[
            pl.BlockSpec((R, D_FEAT), lambda i: (i, 0)),
            pl.BlockSpec((1, R, D_HALF), lambda i: (0, i, 0)),
            pl.BlockSpec((1, R, D_HALF), lambda i: (1, i, 0)),
            pl.BlockSpec((1, R, D_HALF), lambda i: (0, i, 0)),
            pl.BlockSpec((1, R, D_HALF), lambda i: (1, i, 0)),
            pl.BlockSpec((1, R, D_EDGE), lambda i: (0, i, 0)),
            pl.BlockSpec((1, R, D_EDGE), lambda i: (1, i, 0)),
            full((D_HALF, D_FEAT)),
            full((D_HALF, D_FEAT)),
            full((D_EDGE, D_FEAT)),
            full((D_FEAT, D_FEAT)),
            full((D_FEAT, D_FEAT)),
            full((D_FEAT, D_FEAT)),
            full((D_FEAT, D_FEAT)),
            full((1, D_FEAT)),
            full((1, D_FEAT)),
            full((1, D_FEAT)),
            full((1, D_FEAT)),
        ],
        out_specs=pl.BlockSpec((R, D_FEAT), lambda i: (i, 0)),
        out_shape=jax.ShapeDtypeStruct((N_NODES, D_FEAT), jnp.float32),
    )(nodes, pnL, pnL, pnR, pnR, pe, pe,
      W_message[:D_HALF], W_message[D_HALF:D_FEAT], W_message[D_FEAT:],
      W_node, mlp_W1[:D_FEAT], mlp_W1[D_FEAT:], mlp_W2,
      mlp_b1.reshape(1, -1), mlp_b2.reshape(1, -1),
      ln_scale.reshape(1, -1), ln_bias.reshape(1, -1))
    return out


# speedup vs baseline: 5.7383x; 1.5732x over previous
import functools

import jax
import jax.numpy as jnp
from jax import lax
from jax.experimental import pallas as pl
from jax.experimental.pallas import tpu as pltpu
from jax.experimental.pallas import tpu_sc as plsc

N_NODES = 10000
N_EDGES = 320000
D_FEAT = 128
D_HALF = 64
D_EDGE = 16
LN_EPS = 1e-6

NC = 2
NS = 16
NW = NC * NS

CHUNK = 80
EDGES_PER_W = N_EDGES // NW
CHUNKS_PER_W = EDGES_PER_W // CHUNK
N_PAD = 10240
ROWS_PER_TILE = N_PAD // NS


def _sc_body(nodes2_hbm, edges_hbm, s0_hbm, s1_hbm, recv_hbm, z_hbm, ze_hbm,
             pnL_hbm, pnR_hbm, pe_hbm,
             sidx0, sidx1, ridx, rows, erows, acc, eacc,
             gs0, gs1, es0, es1):
    cid = lax.axis_index("c")
    sid = lax.axis_index("s")
    w = cid * NS + sid
    mine = pl.ds(sid * ROWS_PER_TILE, ROWS_PER_TILE)
    gsem = (gs0, gs1)
    esem = (es0, es1)

    pltpu.sync_copy(z_hbm, acc.at[mine])
    pltpu.sync_copy(ze_hbm, eacc.at[mine])

    pltpu.sync_copy(s0_hbm.at[w], sidx0)
    pltpu.sync_copy(s1_hbm.at[w], sidx1)
    pltpu.sync_copy(recv_hbm.at[w], ridx)

    plsc.subcore_barrier()

    def run_pass(sidx, with_edges):
        for b in (0, 1):
            pltpu.async_copy(nodes2_hbm.at[sidx.at[b]], rows.at[b], gsem[b])
            if with_edges:
                pltpu.async_copy(
                    edges_hbm.at[pl.ds(w * EDGES_PER_W + b * CHUNK, CHUNK)],
                    erows.at[b], esem[b])

        @pl.loop(0, CHUNKS_PER_W, step=2)
        def _(j):
            for b in (0, 1):
                c = j + b
                @pl.when(c < CHUNKS_PER_W)
                def _():
                    pltpu.make_async_copy(
                        nodes2_hbm.at[sidx.at[c]], rows.at[b], gsem[b]).wait()
                    pltpu.sync_copy(rows.at[b], acc.at[ridx.at[c]], add=True)
                    if with_edges:
                        pltpu.make_async_copy(
                            edges_hbm.at[pl.ds(w * EDGES_PER_W + c * CHUNK,
                                               CHUNK)],
                            erows.at[b], esem[b]).wait()
                        pltpu.sync_copy(erows.at[b], eacc.at[ridx.at[c]],
                                        add=True)

                    @pl.when(c + 2 < CHUNKS_PER_W)
                    def _():
                        pltpu.async_copy(
                            nodes2_hbm.at[sidx.at[c + 2]], rows.at[b],
                            gsem[b])
                        if with_edges:
                            pltpu.async_copy(
                                edges_hbm.at[
                                    pl.ds(w * EDGES_PER_W + (c + 2) * CHUNK,
                                          CHUNK)],
                                erows.at[b], esem[b])

    run_pass(sidx0, True)
    plsc.subcore_barrier()

    pltpu.sync_copy(acc.at[mine], pnL_hbm.at[cid].at[mine])
    pltpu.sync_copy(eacc.at[mine], pe_hbm.at[cid].at[mine])
    pltpu.sync_copy(z_hbm, acc.at[mine])
    plsc.subcore_barrier()

    run_pass(sidx1, False)
    plsc.subcore_barrier()

    pltpu.sync_copy(acc.at[mine], pnR_hbm.at[cid].at[mine])


_sc_segsum = functools.partial(
    pl.kernel,
    out_type=[
        jax.ShapeDtypeStruct((NC, N_PAD, D_HALF), jnp.float32),
        jax.ShapeDtypeStruct((NC, N_PAD, D_HALF), jnp.float32),
        jax.ShapeDtypeStruct((NC, N_PAD, D_EDGE), jnp.float32),
    ],
    mesh=plsc.VectorSubcoreMesh(core_axis_name="c", subcore_axis_name="s"),
    scratch_types=[
        pltpu.VMEM((CHUNKS_PER_W, CHUNK), jnp.int32),
        pltpu.VMEM((CHUNKS_PER_W, CHUNK), jnp.int32),
        pltpu.VMEM((CHUNKS_PER_W, CHUNK), jnp.int32),
        pltpu.VMEM((2, CHUNK, D_HALF), jnp.float32),
        pltpu.VMEM((2, CHUNK, D_EDGE), jnp.float32),
        pltpu.VMEM_SHARED((N_PAD, D_HALF), jnp.float32),
        pltpu.VMEM_SHARED((N_PAD, D_EDGE), jnp.float32),
        pltpu.SemaphoreType.DMA,
        pltpu.SemaphoreType.DMA,
        pltpu.SemaphoreType.DMA,
        pltpu.SemaphoreType.DMA,
    ],
    compiler_params=pltpu.CompilerParams(use_tc_tiling_on_sc=False),
)(_sc_body)


def _tc_body(nodes, pl0, pl1, pr0, pr1, pe0, pe1,
             wm_a, wm_b, wm_bot, w_node, w1_top, w1_bot, w2,
             b1, b2, g, b, out):
    f32 = jnp.float32
    agg = (jnp.dot(pl0[0] + pl1[0], wm_a[...], preferred_element_type=f32)
           + jnp.dot(pr0[0] + pr1[0], wm_b[...], preferred_element_type=f32)
           + jnp.dot(pe0[0] + pe1[0], wm_bot[...], preferred_element_type=f32))
    x = nodes[...]
    h = jnp.maximum(
        jnp.dot(x, w1_top[...], preferred_element_type=f32)
        + jnp.dot(agg, w1_bot[...], preferred_element_type=f32)
        + b1[...], 0.0)
    pre = (jnp.dot(h, w2[...], preferred_element_type=f32) + b2[...]
           + jnp.dot(x, w_node[...], preferred_element_type=f32))
    mean = jnp.mean(pre, axis=-1, keepdims=True)
    c = pre - mean
    var = jnp.mean(c * c, axis=-1, keepdims=True)
    out[...] = c * lax.rsqrt(var + LN_EPS) * g[...] + b[...]


def kernel(nodes, edges, W_message, W_node, mlp_W1, mlp_b1, mlp_W2, mlp_b2,
           ln_scale, ln_bias, senders, receivers):
    nodes2 = nodes.reshape(2 * N_NODES, D_HALF)
    s0 = (senders * 2).reshape(NW, CHUNKS_PER_W, CHUNK)
    s1 = (senders * 2 + 1).reshape(NW, CHUNKS_PER_W, CHUNK)
    recv3d = receivers.reshape(NW, CHUNKS_PER_W, CHUNK)
    z = jnp.zeros((ROWS_PER_TILE, D_HALF), jnp.float32)
    ze = jnp.zeros((ROWS_PER_TILE, D_EDGE), jnp.float32)

    pnL, pnR, pe = _sc_segsum(nodes2, edges, s0, s1, recv3d, z, ze)

    R = 1000
    grid = (N_NODES // R,)
    full = lambda shape: pl.BlockSpec(shape, lambda i: (0, 0))
    out = pl.pallas_call(
        _tc_body,
        grid=grid,
        in_specs=[
            pl.BlockSpec((R, D_FEAT), lambda i: (i, 0)),
            pl.BlockSpec((1, R, D_HALF), lambda i: (0, i, 0)),
            pl.BlockSpec((1, R, D_HALF), lambda i: (1, i, 0)),
            pl.BlockSpec((1, R, D_HALF), lambda i: (0, i, 0)),
            pl.BlockSpec((1, R, D_HALF), lambda i: (1, i, 0)),
            pl.BlockSpec((1, R, D_EDGE), lambda i: (0, i, 0)),
            pl.BlockSpec((1, R, D_EDGE), lambda i: (1, i, 0)),
            full((D_HALF, D_FEAT)),
            full((D_HALF, D_FEAT)),
            full((D_EDGE, D_FEAT)),
            full((D_FEAT, D_FEAT)),
            full((D_FEAT, D_FEAT)),
            full((D_FEAT, D_FEAT)),
            full((D_FEAT, D_FEAT)),
            full((1, D_FEAT)),
            full((1, D_FEAT)),
            full((1, D_FEAT)),
            full((1, D_FEAT)),
        ],
        out_specs=pl.BlockSpec((R, D_FEAT), lambda i: (i, 0)),
        out_shape=jax.ShapeDtypeStruct((N_NODES, D_FEAT), jnp.float32),
    )(nodes, pnL, pnL, pnR, pnR, pe, pe,
      W_message[:D_HALF], W_message[D_HALF:D_FEAT], W_message[D_FEAT:],
      W_node, mlp_W1[:D_FEAT], mlp_W1[D_FEAT:], mlp_W2,
      mlp_b1.reshape(1, -1), mlp_b2.reshape(1, -1),
      ln_scale.reshape(1, -1), ln_bias.reshape(1, -1))
    return out
